# baseline (speedup 1.0000x reference)
import functools

import jax
import jax.numpy as jnp
from jax import lax
from jax.experimental import pallas as pl
from jax.experimental.pallas import tpu as pltpu
from jax.experimental.pallas import tpu_sc as plsc

B = 16384
D = 16
NW = 32
BPW = B // NW
CHUNK = 128
NCHUNK = BPW // CHUNK

NROWS = 1_000_000
NPACK = NROWS // 8
TC_C = 2048


_LOG2E = 1.4426950408889634
_LN2 = 0.6931471805599453
_EXP_C = (1 / 5040, 1 / 720, 1 / 120, 1 / 24, 1 / 6, 0.5, 1.0, 1.0)


def _exp(y):
    t = jnp.clip(y * _LOG2E, -126.0, 126.0)
    n = t.astype(jnp.int32)
    f = t - n.astype(jnp.float32)
    u = f * _LN2
    p = jnp.float32(_EXP_C[0])
    for c in _EXP_C[1:]:
        p = p * u + jnp.float32(c)
    scale = lax.bitcast_convert_type((n + 127) << 23, jnp.float32)
    return p * scale


def _sigmoid(x):
    den = 1.0 + _exp(-x)
    r = 1.0 / den
    return r * (2.0 - den * r)


NCOL = NROWS // 128
COLS_PW = (NCOL + NW - 1) // NW
TAIL = NROWS - NCOL * 128


@functools.partial(
    pl.kernel,
    out_type=[jax.ShapeDtypeStruct((NROWS * D,), jnp.float32),
              jax.ShapeDtypeStruct((NROWS * D,), jnp.float32)],
    mesh=plsc.VectorSubcoreMesh(core_axis_name="c", subcore_axis_name="s"),
    compiler_params=pltpu.CompilerParams(needs_layout_passes=False),
    scratch_types=[
        pltpu.VMEM((D, 128), jnp.float32),
        pltpu.VMEM((16 * 128,), jnp.float32),
    ],
)
def _repack_sc(utT_hbm, itT_hbm, up_hbm, ip_hbm, xb, pb):
    wid = lax.axis_index("s") * 2 + lax.axis_index("c")
    lane = lax.iota(jnp.int32, 16)

    def pack_block(src_hbm, dst_hbm, c, width):
        off = pl.multiple_of(c * 128, 128)
        pltpu.sync_copy(src_hbm.at[:, pl.ds(off, 128)], xb)
        for a in range(width // 8):
            for k in range(8):
                col = jnp.full((16,), 8 * a + k, jnp.int32)
                seg = plsc.load_gather(xb, [lane, col])
                pb[pl.ds(a * 128 + k * 16, 16)] = seg
        pltpu.sync_copy(pb.at[pl.ds(0, width * D)],
                        dst_hbm.at[pl.ds(c * 2048, width * D)])

    def col_body(t, _):
        c = wid + t * NW

        @pl.when(c < NCOL)
        def _():
            pack_block(utT_hbm, up_hbm, c, 128)
            pack_block(itT_hbm, ip_hbm, c, 128)
        return _

    lax.fori_loop(0, COLS_PW, col_body, None)


@functools.partial(
    pl.kernel,
    out_type=jax.ShapeDtypeStruct((B, 2), jnp.float32),
    mesh=plsc.VectorSubcoreMesh(core_axis_name="c", subcore_axis_name="s"),
    compiler_params=pltpu.CompilerParams(needs_layout_passes=False),
    scratch_types=[
        pltpu.VMEM((BPW,), jnp.int32),
        pltpu.VMEM((BPW,), jnp.int32),
        pltpu.VMEM((BPW,), jnp.int32),
        pltpu.VMEM((BPW,), jnp.int32),
        pltpu.VMEM((BPW,), jnp.int32),
        pltpu.VMEM((BPW,), jnp.int32),
        pltpu.VMEM((CHUNK, 128), jnp.float32),
        pltpu.VMEM((CHUNK, 128), jnp.float32),
        pltpu.VMEM((CHUNK, 128), jnp.float32),
        pltpu.VMEM((8, 128), jnp.float32),
        pltpu.VMEM((8, 128), jnp.float32),
        pltpu.VMEM((D,), jnp.float32),
        pltpu.VMEM((16,), jnp.float32),
        pltpu.VMEM((BPW, 2), jnp.float32),
        pltpu.SemaphoreType.DMA,
    ],
)
def _lightgcn_sc(user_hbm, pos_hbm, neg_hbm, ut_hbm, it_hbm, auxu_hbm,
                 auxi_hbm, w_hbm, b_hbm, out_hbm, idx_u, idx_p, idx_n,
                 hid_u, hid_p, hid_n, rows_u, rows_p, rows_n, auxu_v, auxi_v,
                 w_v, b_v, out_v, sem):
    wid = lax.axis_index("s") * 2 + lax.axis_index("c")
    base = wid * BPW

    pltpu.sync_copy(user_hbm.at[pl.ds(base, BPW)], idx_u)
    pltpu.sync_copy(pos_hbm.at[pl.ds(base, BPW)], idx_p)
    pltpu.sync_copy(neg_hbm.at[pl.ds(base, BPW)], idx_n)
    pltpu.sync_copy(auxu_hbm, auxu_v)
    pltpu.sync_copy(auxi_hbm, auxi_v)
    pltpu.sync_copy(w_hbm, w_v)
    pltpu.sync_copy(b_hbm, b_v)

    def hbuild(g, _):
        sl = pl.ds(g * 16, 16)
        hid_u[sl] = idx_u[sl] >> 3
        hid_p[sl] = idx_p[sl] >> 3
        hid_n[sl] = idx_n[sl] >> 3
        return _

    lax.fori_loop(0, BPW // 16, hbuild, None)

    lane = lax.iota(jnp.int32, 16)
    col0 = jnp.zeros((16,), jnp.int32)
    col1 = jnp.ones((16,), jnp.int32)
    bias_vec = b_v[...]
    def _mxu_round(x):
        i = lax.bitcast_convert_type(x, jnp.int32)
        r = (i + 0x7FFF + ((i >> 16) & 1)) & jnp.int32(-65536)
        return lax.bitcast_convert_type(r, jnp.float32)

    wvec = _mxu_round(w_v[...])

    for j in range(NCHUNK):
        sl = pl.ds(j * CHUNK, CHUNK)
        cps = [
            pltpu.async_copy(ut_hbm.at[hid_u.at[sl]], rows_u, sem),
            pltpu.async_copy(it_hbm.at[hid_p.at[sl]], rows_p, sem),
            pltpu.async_copy(it_hbm.at[hid_n.at[sl]], rows_n, sem),
        ]
        for cp in cps:
            cp.wait()

        def block_body(lb, _, j=j):
            blk = j * (CHUNK // 16) + lb
            rows = blk * 16 + lane
            loc = lb * 16 + lane
            rv_u = idx_u[pl.ds(blk * 16, 16)]
            rv_p = idx_p[pl.ds(blk * 16, 16)]
            rv_n = idx_n[pl.ds(blk * 16, 16)]
            su = (rv_u & 7) * 16
            sp = (rv_p & 7) * 16
            sn = (rv_n & 7) * 16
            tm_u = rv_u >= NCOL * 128
            tm_p = rv_p >= NCOL * 128
            tm_n = rv_n >= NCOL * 128
            th_u = jnp.maximum((rv_u - NCOL * 128) >> 3, 0)
            th_p = jnp.maximum((rv_p - NCOL * 128) >> 3, 0)
            th_n = jnp.maximum((rv_n - NCOL * 128) >> 3, 0)
            pos_acc = bias_vec
            neg_acc = bias_vec
            for d in range(D):
                u = plsc.load_gather(rows_u, [loc, su + d])
                p = plsc.load_gather(rows_p, [loc, sp + d])
                n = plsc.load_gather(rows_n, [loc, sn + d])
                u = jnp.where(tm_u, plsc.load_gather(auxu_v, [th_u, su + d]), u)
                p = jnp.where(tm_p, plsc.load_gather(auxi_v, [th_p, sp + d]), p)
                n = jnp.where(tm_n, plsc.load_gather(auxi_v, [th_n, sn + d]), n)
                wd = wvec[d]
                pos_acc = pos_acc + _mxu_round(_sigmoid(u * p)) * wd
                neg_acc = neg_acc + _mxu_round(_sigmoid(u * n)) * wd
            plsc.store_scatter(out_v, [rows, col0], pos_acc)
            plsc.store_scatter(out_v, [rows, col1], neg_acc)
            return _

        lax.fori_loop(0, CHUNK // 16, block_body, None)

    pltpu.sync_copy(out_v, out_hbm.at[pl.ds(base, BPW)])


def kernel(user, pos, neg, user_table, item_table, W, b):
    user = jnp.asarray(user, jnp.int32).reshape(B)
    pos = jnp.asarray(pos, jnp.int32).reshape(B)
    neg = jnp.asarray(neg, jnp.int32).reshape(B)
    w = W.reshape(D)
    b16 = jnp.broadcast_to(b.reshape(1), (16,)).astype(jnp.float32)
    ut_flat, it_flat = _repack_sc(user_table.T, item_table.T)
    ut_pk = ut_flat.reshape(NPACK, 128)
    it_pk = it_flat.reshape(NPACK, 128)
    aux_u = user_table[NCOL * 128:].reshape(8, 128)
    aux_i = item_table[NCOL * 128:].reshape(8, 128)
    return _lightgcn_sc(user, pos, neg, ut_pk, it_pk, aux_u, aux_i, w, b16)


# speedup vs baseline: 1.5826x; 1.5826x over previous
import functools

import jax
import jax.numpy as jnp
from jax import lax
from jax.experimental import pallas as pl
from jax.experimental.pallas import tpu as pltpu
from jax.experimental.pallas import tpu_sc as plsc

B = 16384
D = 16
NW = 32
BPW = B // NW
CHUNK = 128
NCHUNK = BPW // CHUNK

NROWS = 1_000_000
NPACK = NROWS // 8
TC_C = 2048


_LOG2E = 1.4426950408889634
_LN2 = 0.6931471805599453
_EXP_C = (1 / 5040, 1 / 720, 1 / 120, 1 / 24, 1 / 6, 0.5, 1.0, 1.0)


def _exp(y):
    t = jnp.clip(y * _LOG2E, -126.0, 126.0)
    n = t.astype(jnp.int32)
    f = t - n.astype(jnp.float32)
    u = f * _LN2
    p = jnp.float32(_EXP_C[0])
    for c in _EXP_C[1:]:
        p = p * u + jnp.float32(c)
    scale = lax.bitcast_convert_type((n + 127) << 23, jnp.float32)
    return p * scale


def _sigmoid(x):
    den = 1.0 + _exp(-x)
    r = 1.0 / den
    return r * (2.0 - den * r)


NCOL = NROWS // 128
COLS_PW = (NCOL + NW - 1) // NW
TAIL = NROWS - NCOL * 128


@functools.partial(
    pl.kernel,
    out_type=[jax.ShapeDtypeStruct((NROWS * D,), jnp.float32),
              jax.ShapeDtypeStruct((NROWS * D,), jnp.float32)],
    mesh=plsc.VectorSubcoreMesh(core_axis_name="c", subcore_axis_name="s"),
    compiler_params=pltpu.CompilerParams(needs_layout_passes=False),
    scratch_types=[
        pltpu.VMEM((D, 128), jnp.float32),
        pltpu.VMEM((16 * 128,), jnp.float32),
    ],
)
def _repack_sc(utT_hbm, itT_hbm, up_hbm, ip_hbm, xb, pb):
    wid = lax.axis_index("s") * 2 + lax.axis_index("c")
    lane = lax.iota(jnp.int32, 16)

    def pack_block(src_hbm, dst_hbm, c, width):
        off = pl.multiple_of(c * 128, 128)
        pltpu.sync_copy(src_hbm.at[:, pl.ds(off, 128)], xb)
        for a in range(width // 8):
            for k in range(8):
                col = jnp.full((16,), 8 * a + k, jnp.int32)
                seg = plsc.load_gather(xb, [lane, col])
                pb[pl.ds(a * 128 + k * 16, 16)] = seg
        pltpu.sync_copy(pb.at[pl.ds(0, width * D)],
                        dst_hbm.at[pl.ds(c * 2048, width * D)])

    def col_body(t, _):
        c = wid + t * NW

        @pl.when(c < NCOL)
        def _():
            pack_block(utT_hbm, up_hbm, c, 128)
            pack_block(itT_hbm, ip_hbm, c, 128)
        return _

    lax.fori_loop(0, COLS_PW, col_body, None)


@functools.partial(
    pl.kernel,
    out_type=jax.ShapeDtypeStruct((B, 2), jnp.float32),
    mesh=plsc.VectorSubcoreMesh(core_axis_name="c", subcore_axis_name="s"),
    compiler_params=pltpu.CompilerParams(needs_layout_passes=False),
    scratch_types=[
        pltpu.VMEM((BPW,), jnp.int32),
        pltpu.VMEM((BPW,), jnp.int32),
        pltpu.VMEM((BPW,), jnp.int32),
        pltpu.VMEM((BPW,), jnp.int32),
        pltpu.VMEM((BPW,), jnp.int32),
        pltpu.VMEM((BPW,), jnp.int32),
        pltpu.VMEM((CHUNK, 128), jnp.float32),
        pltpu.VMEM((CHUNK, 128), jnp.float32),
        pltpu.VMEM((CHUNK, 128), jnp.float32),
        pltpu.VMEM((8, 128), jnp.float32),
        pltpu.VMEM((8, 128), jnp.float32),
        pltpu.VMEM((D,), jnp.float32),
        pltpu.VMEM((16,), jnp.float32),
        pltpu.VMEM((BPW, 2), jnp.float32),
        pltpu.SemaphoreType.DMA,
    ],
)
def _lightgcn_sc(user_hbm, pos_hbm, neg_hbm, ut_hbm, it_hbm, auxu_hbm,
                 auxi_hbm, w_hbm, b_hbm, out_hbm, idx_u, idx_p, idx_n,
                 hid_u, hid_p, hid_n, rows_u, rows_p, rows_n, auxu_v, auxi_v,
                 w_v, b_v, out_v, sem):
    wid = lax.axis_index("s") * 2 + lax.axis_index("c")
    base = wid * BPW

    pltpu.sync_copy(user_hbm.at[pl.ds(base, BPW)], idx_u)
    pltpu.sync_copy(pos_hbm.at[pl.ds(base, BPW)], idx_p)
    pltpu.sync_copy(neg_hbm.at[pl.ds(base, BPW)], idx_n)
    pltpu.sync_copy(auxu_hbm, auxu_v)
    pltpu.sync_copy(auxi_hbm, auxi_v)
    pltpu.sync_copy(w_hbm, w_v)
    pltpu.sync_copy(b_hbm, b_v)

    def hbuild(g, _):
        sl = pl.ds(g * 16, 16)
        hid_u[sl] = idx_u[sl] >> 3
        hid_p[sl] = idx_p[sl] >> 3
        hid_n[sl] = idx_n[sl] >> 3
        return _

    lax.fori_loop(0, BPW // 16, hbuild, None)

    lane = lax.iota(jnp.int32, 16)
    col0 = jnp.zeros((16,), jnp.int32)
    col1 = jnp.ones((16,), jnp.int32)
    bias_vec = b_v[...]
    def _mxu_round(x):
        i = lax.bitcast_convert_type(x, jnp.int32)
        r = (i + 0x7FFF + ((i >> 16) & 1)) & jnp.int32(-65536)
        return lax.bitcast_convert_type(r, jnp.float32)

    wvec = _mxu_round(w_v[...])

    for j in range(NCHUNK):
        sl = pl.ds(j * CHUNK, CHUNK)
        cps = [
            pltpu.async_copy(ut_hbm.at[hid_u.at[sl]], rows_u, sem),
            pltpu.async_copy(it_hbm.at[hid_p.at[sl]], rows_p, sem),
            pltpu.async_copy(it_hbm.at[hid_n.at[sl]], rows_n, sem),
        ]
        for cp in cps:
            cp.wait()

        def block_body(lb, _, j=j):
            blk = j * (CHUNK // 16) + lb
            rows = blk * 16 + lane
            loc = lb * 16 + lane
            rv_u = idx_u[pl.ds(blk * 16, 16)]
            rv_p = idx_p[pl.ds(blk * 16, 16)]
            rv_n = idx_n[pl.ds(blk * 16, 16)]
            su = (rv_u & 7) * 16
            sp = (rv_p & 7) * 16
            sn = (rv_n & 7) * 16
            tm_u = rv_u >= NCOL * 128
            tm_p = rv_p >= NCOL * 128
            tm_n = rv_n >= NCOL * 128
            th_u = jnp.maximum((rv_u - NCOL * 128) >> 3, 0)
            th_p = jnp.maximum((rv_p - NCOL * 128) >> 3, 0)
            th_n = jnp.maximum((rv_n - NCOL * 128) >> 3, 0)
            pos_acc = bias_vec
            neg_acc = bias_vec
            for d in range(D):
                u = plsc.load_gather(rows_u, [loc, su + d])
                p = plsc.load_gather(rows_p, [loc, sp + d])
                n = plsc.load_gather(rows_n, [loc, sn + d])
                u = jnp.where(tm_u, plsc.load_gather(auxu_v, [th_u, su + d]), u)
                p = jnp.where(tm_p, plsc.load_gather(auxi_v, [th_p, sp + d]), p)
                n = jnp.where(tm_n, plsc.load_gather(auxi_v, [th_n, sn + d]), n)
                wd = wvec[d]
                pos_acc = pos_acc + _mxu_round(_sigmoid(u * p)) * wd
                neg_acc = neg_acc + _mxu_round(_sigmoid(u * n)) * wd
            plsc.store_scatter(out_v, [rows, col0], pos_acc)
            plsc.store_scatter(out_v, [rows, col1], neg_acc)
            return _

        lax.fori_loop(0, CHUNK // 16, block_body, None)

    pltpu.sync_copy(out_v, out_hbm.at[pl.ds(base, BPW)])


def kernel(user, pos, neg, user_table, item_table, W, b):
    user = jnp.asarray(user, jnp.int32).reshape(B)
    pos = jnp.asarray(pos, jnp.int32).reshape(B)
    neg = jnp.asarray(neg, jnp.int32).reshape(B)
    w = W.reshape(D)
    b16 = jnp.broadcast_to(b.reshape(1), (16,)).astype(jnp.float32)
    ut_pk = user_table.reshape(NPACK, 128)
    it_pk = item_table.reshape(NPACK, 128)
    aux_u = user_table[NCOL * 128:].reshape(8, 128)
    aux_i = item_table[NCOL * 128:].reshape(8, 128)
    return _lightgcn_sc(user, pos, neg, ut_pk, it_pk, aux_u, aux_i, w, b16)


# speedup vs baseline: 1.6200x; 1.0236x over previous
import functools

import jax
import jax.numpy as jnp
from jax import lax
from jax.experimental import pallas as pl
from jax.experimental.pallas import tpu as pltpu
from jax.experimental.pallas import tpu_sc as plsc

B = 16384
D = 16
NW = 32
BPW = B // NW
CHUNK = 128
NCHUNK = BPW // CHUNK


_LOG2E = 1.4426950408889634
_LN2 = 0.6931471805599453
_EXP_C = (1 / 5040, 1 / 720, 1 / 120, 1 / 24, 1 / 6, 0.5, 1.0, 1.0)


def _exp(y):
    t = jnp.clip(y * _LOG2E, -126.0, 126.0)
    n = t.astype(jnp.int32)
    f = t - n.astype(jnp.float32)
    u = f * _LN2
    p = jnp.float32(_EXP_C[0])
    for c in _EXP_C[1:]:
        p = p * u + jnp.float32(c)
    scale = lax.bitcast_convert_type((n + 127) << 23, jnp.float32)
    return p * scale


def _sigmoid(x):
    den = 1.0 + _exp(-x)
    r = 1.0 / den
    return r * (2.0 - den * r)


@functools.partial(
    pl.kernel,
    out_type=jax.ShapeDtypeStruct((B, 2), jnp.float32),
    mesh=plsc.VectorSubcoreMesh(core_axis_name="c", subcore_axis_name="s"),
    compiler_params=pltpu.CompilerParams(
        needs_layout_passes=False, use_tc_tiling_on_sc=False),
    scratch_types=[
        pltpu.VMEM((BPW,), jnp.int32),
        pltpu.VMEM((BPW,), jnp.int32),
        pltpu.VMEM((BPW,), jnp.int32),
        pltpu.VMEM((BPW, D), jnp.float32),
        pltpu.VMEM((BPW, D), jnp.float32),
        pltpu.VMEM((BPW, D), jnp.float32),
        pltpu.VMEM((D,), jnp.float32),
        pltpu.VMEM((16,), jnp.float32),
        pltpu.VMEM((BPW, 2), jnp.float32),
        pltpu.SemaphoreType.DMA,
    ],
)
def _lightgcn_sc(user_hbm, pos_hbm, neg_hbm, ut_hbm, it_hbm, w_hbm, b_hbm,
                 out_hbm, idx_u, idx_p, idx_n, rows_u, rows_p, rows_n,
                 w_v, b_v, out_v, sem):
    wid = lax.axis_index("s") * 2 + lax.axis_index("c")
    base = wid * BPW

    pltpu.sync_copy(user_hbm.at[pl.ds(base, BPW)], idx_u)
    pltpu.sync_copy(pos_hbm.at[pl.ds(base, BPW)], idx_p)
    pltpu.sync_copy(neg_hbm.at[pl.ds(base, BPW)], idx_n)
    pltpu.sync_copy(w_hbm, w_v)
    pltpu.sync_copy(b_hbm, b_v)

    copies = []
    for j in range(NCHUNK):
        sl = pl.ds(j * CHUNK, CHUNK)
        copies.append(pltpu.async_copy(ut_hbm.at[idx_u.at[sl]], rows_u.at[sl], sem))
        copies.append(pltpu.async_copy(it_hbm.at[idx_p.at[sl]], rows_p.at[sl], sem))
        copies.append(pltpu.async_copy(it_hbm.at[idx_n.at[sl]], rows_n.at[sl], sem))
    for cp in copies:
        cp.wait()

    lane = lax.iota(jnp.int32, 16)
    col0 = jnp.zeros((16,), jnp.int32)
    col1 = jnp.ones((16,), jnp.int32)
    bias_vec = b_v[...]
    def _mxu_round(x):
        i = lax.bitcast_convert_type(x, jnp.int32)
        r = (i + 0x7FFF + ((i >> 16) & 1)) & jnp.int32(-65536)
        return lax.bitcast_convert_type(r, jnp.float32)

    wvec = _mxu_round(w_v[...])

    def block_body(blk, _):
        rows = blk * 16 + lane
        pos_acc = bias_vec
        neg_acc = bias_vec
        for d in range(D):
            cold = jnp.full((16,), d, jnp.int32)
            u = plsc.load_gather(rows_u, [rows, cold])
            p = plsc.load_gather(rows_p, [rows, cold])
            n = plsc.load_gather(rows_n, [rows, cold])
            wd = wvec[d]
            pos_acc = pos_acc + _mxu_round(_sigmoid(u * p)) * wd
            neg_acc = neg_acc + _mxu_round(_sigmoid(u * n)) * wd
        plsc.store_scatter(out_v, [rows, col0], pos_acc)
        plsc.store_scatter(out_v, [rows, col1], neg_acc)
        return _

    lax.fori_loop(0, BPW // 16, block_body, None)

    pltpu.sync_copy(out_v, out_hbm.at[pl.ds(base, BPW)])


def kernel(user, pos, neg, user_table, item_table, W, b):
    user = jnp.asarray(user, jnp.int32).reshape(B)
    pos = jnp.asarray(pos, jnp.int32).reshape(B)
    neg = jnp.asarray(neg, jnp.int32).reshape(B)
    w = W.reshape(D)
    b16 = jnp.broadcast_to(b.reshape(1), (16,)).astype(jnp.float32)
    return _lightgcn_sc(user, pos, neg, user_table, item_table, w, b16)
